# Initial kernel scaffold; baseline (speedup 1.0000x reference)
import jax
import jax.numpy as jnp
from jax.experimental import pallas as pl


def kernel(conv1_w, bn1, s0b0_conv1, s0b0_bn1, s0b0_conv2, s0b0_bn2, s0b0_conv3, s0b0_bn3, s0b0_dconv, s0b0_dbn, s0b1_conv1, s0b1_bn1, s0b1_conv2, s0b1_bn2, s0b1_conv3, s0b1_bn3, s0b2_conv1, s0b2_bn1, s0b2_conv2, s0b2_bn2, s0b2_conv3, s0b2_bn3, s1b0_conv1, s1b0_bn1, s1b0_conv2, s1b0_bn2, s1b0_conv3, s1b0_bn3, s1b0_dconv, s1b0_dbn, s1b1_conv1, s1b1_bn1, s1b1_conv2, s1b1_bn2, s1b1_conv3, s1b1_bn3, s1b2_conv1, s1b2_bn1, s1b2_conv2, s1b2_bn2, s1b2_conv3, s1b2_bn3, s1b3_conv1, s1b3_bn1, s1b3_conv2, s1b3_bn2, s1b3_conv3, s1b3_bn3, s2b0_conv1, s2b0_bn1, s2b0_conv2, s2b0_bn2, s2b0_conv3, s2b0_bn3, s2b0_dconv, s2b0_dbn, s2b1_conv1, s2b1_bn1, s2b1_conv2, s2b1_bn2, s2b1_conv3, s2b1_bn3, s2b2_conv1, s2b2_bn1, s2b2_conv2, s2b2_bn2, s2b2_conv3, s2b2_bn3, s2b3_conv1, s2b3_bn1, s2b3_conv2, s2b3_bn2, s2b3_conv3, s2b3_bn3, s2b4_conv1, s2b4_bn1, s2b4_conv2, s2b4_bn2, s2b4_conv3, s2b4_bn3, s2b5_conv1, s2b5_bn1, s2b5_conv2, s2b5_bn2, s2b5_conv3, s2b5_bn3, s3b0_conv1, s3b0_bn1, s3b0_conv2, s3b0_bn2, s3b0_conv3, s3b0_bn3, s3b0_dconv, s3b0_dbn, s3b1_conv1, s3b1_bn1, s3b1_conv2, s3b1_bn2, s3b1_conv3, s3b1_bn3, s3b2_conv1, s3b2_bn1, s3b2_conv2, s3b2_bn2, s3b2_conv3, s3b2_bn3, fc_w, fc_b, x):
    raise NotImplementedError("write your pallas kernel here")


# speedup vs baseline: 3.8590x; 3.8590x over previous
import functools

import jax
import jax.numpy as jnp
from jax.experimental import pallas as pl
from jax.experimental.pallas import tpu as pltpu

_BF16 = jnp.bfloat16
_F32 = jnp.float32

_STAGES = ((3, 64, 256, 1), (4, 128, 512, 2), (6, 256, 1024, 2), (3, 512, 2048, 2))
_SUBBATCH = (1, 2, 4, 8)


def _pad_hw1(y):
    nb, h, w, c = y.shape
    zw = jnp.zeros((nb, h, 1, c), y.dtype)
    y = jnp.concatenate([zw, y, zw], axis=2)
    zh = jnp.zeros((nb, 1, w + 2, c), y.dtype)
    return jnp.concatenate([zh, y, zh], axis=1)


def _tap(a, oh, ow, ho, wo, stride):
    nb = a.shape[0]
    c = a.shape[3]
    v = a[:, oh:oh + stride * ho, ow:ow + stride * wo, :]
    if stride > 1:
        v = v.reshape(nb, ho, stride, wo, stride, c)[:, :, 0, :, 0, :]
    return v


def _block_body(refs, *, stride, has_down):
    if has_down:
        (x_ref, w1_ref, w2_ref, w3_ref, wd_ref, a1_ref, a2_ref, a3_ref,
         ad_ref, o_ref) = refs
    else:
        x_ref, w1_ref, w2_ref, w3_ref, a1_ref, a2_ref, a3_ref, o_ref = refs
        wd_ref = ad_ref = None

    nb, h, w, cin = x_ref.shape
    cmid = w1_ref.shape[1]
    cout = w3_ref.shape[1]
    ho, wo = h // stride, w // stride

    x = x_ref[...]
    xf = x.reshape(nb * h * w, cin)

    y = jnp.dot(xf, w1_ref[...], preferred_element_type=_F32)
    y = jnp.maximum(y * a1_ref[0:1, :] + a1_ref[1:2, :], 0.0).astype(_BF16)

    yp = _pad_hw1(y.reshape(nb, h, w, cmid))
    acc = jnp.zeros((nb * ho * wo, cmid), _F32)
    for kh in range(3):
        for kw in range(3):
            v = _tap(yp, kh, kw, ho, wo, stride).reshape(nb * ho * wo, cmid)
            acc = acc + jnp.dot(v, w2_ref[kh, kw], preferred_element_type=_F32)
    y2 = jnp.maximum(acc * a2_ref[0:1, :] + a2_ref[1:2, :], 0.0).astype(_BF16)

    if has_down:
        xs = _tap(x, 0, 0, ho, wo, stride).reshape(nb * ho * wo, cin)
        idn = jnp.dot(xs, wd_ref[...], preferred_element_type=_F32)
        idn = (idn * ad_ref[0:1, :] + ad_ref[1:2, :]).astype(_BF16)
    else:
        idn = xf

    y3 = jnp.dot(y2, w3_ref[...], preferred_element_type=_F32)
    y3 = y3 * a3_ref[0:1, :] + a3_ref[1:2, :] + idn.astype(_F32)
    y3 = jnp.maximum(y3, 0.0)
    o_ref[...] = y3.astype(_BF16).reshape(nb, ho, wo, cout)


def _bottleneck(x, w1, a1, w2, a2, w3, a3, wd, ad, stride, nb):
    n, h, w, cin = x.shape
    cmid = w1.shape[1]
    cout = w3.shape[1]
    ho, wo = h // stride, w // stride
    has_down = wd is not None

    def batch_map(rank):
        return lambda i: (i,) + (0,) * (rank - 1)

    def const_map(rank):
        return lambda i: (0,) * rank

    in_specs = [pl.BlockSpec((nb, h, w, cin), batch_map(4))]
    args = [x]
    for wt in (w1, w2, w3) + ((wd,) if has_down else ()):
        in_specs.append(pl.BlockSpec(wt.shape, const_map(wt.ndim)))
        args.append(wt)
    for af in (a1, a2, a3) + ((ad,) if has_down else ()):
        in_specs.append(pl.BlockSpec(af.shape, const_map(2)))
        args.append(af)

    body = functools.partial(_block_body, stride=stride, has_down=has_down)

    def wrapped(*refs):
        body(refs)

    return pl.pallas_call(
        wrapped,
        out_shape=jax.ShapeDtypeStruct((n, ho, wo, cout), _BF16),
        grid=(n // nb,),
        in_specs=in_specs,
        out_specs=pl.BlockSpec((nb, ho, wo, cout), batch_map(4)),
        compiler_params=pltpu.CompilerParams(
            dimension_semantics=("arbitrary",),
            vmem_limit_bytes=60 * 1024 * 1024,
        ),
    )(*args)


def _stem_mm_body(a_ref, w_ref, ab_ref, o_ref):
    y = jnp.dot(a_ref[...], w_ref[...], preferred_element_type=_F32)
    o_ref[...] = jnp.maximum(
        y * ab_ref[0:1, :] + ab_ref[1:2, :], 0.0).astype(_BF16)


def _stem(x, w, ab):
    n, h, w_in, c = x.shape
    ho = wo = h // 2
    xp = jnp.pad(x, ((0, 0), (3, 3), (3, 3), (0, 0)))
    taps = [xp[:, i:i + 2 * ho:2, j:j + 2 * wo:2, :]
            for i in range(7) for j in range(7)]
    cols = jnp.concatenate(taps, axis=-1).reshape(n * ho * wo, 49 * c)
    kp = w.shape[0]
    cols = jnp.pad(cols, ((0, 0), (0, kp - 49 * c)))
    m = n * ho * wo
    tm = 512
    cout = 64
    wt = w[:, :cout]
    out = pl.pallas_call(
        _stem_mm_body,
        out_shape=jax.ShapeDtypeStruct((m, cout), _BF16),
        grid=(m // tm,),
        in_specs=[
            pl.BlockSpec((tm, kp), lambda i: (i, 0)),
            pl.BlockSpec((kp, cout), lambda i: (0, 0)),
            pl.BlockSpec((2, cout), lambda i: (0, 0)),
        ],
        out_specs=pl.BlockSpec((tm, cout), lambda i: (i, 0)),
        compiler_params=pltpu.CompilerParams(
            dimension_semantics=("arbitrary",),
            vmem_limit_bytes=48 * 1024 * 1024,
        ),
    )(cols, wt, ab[:, :cout])
    return out.reshape(n, ho, wo, cout)


def _head_body(x_ref, w_ref, b_ref, o_ref):
    x = x_ref[...].astype(_F32)
    nb, h, w, c = x.shape
    feat = jnp.mean(x.reshape(nb, h * w, c), axis=1)
    y = jnp.dot(feat, w_ref[...], preferred_element_type=_F32) + b_ref[...]
    o_ref[...] = jax.nn.sigmoid(y)


def _head(x, fc_w, fc_b):
    n = x.shape[0]
    np_ = fc_w.shape[1]
    out = pl.pallas_call(
        _head_body,
        out_shape=jax.ShapeDtypeStruct((n, np_), _F32),
        compiler_params=pltpu.CompilerParams(
            vmem_limit_bytes=32 * 1024 * 1024),
    )(x, fc_w, fc_b)
    return out[:, :3]


def _maxpool(x):
    return jax.lax.reduce_window(
        x, jnp.array(-jnp.inf, x.dtype), jax.lax.max,
        (1, 3, 3, 1), (1, 2, 2, 1), ((0, 0), (1, 1), (1, 1), (0, 0)))


def _prep_block(cin, cmid, cout, conv1, bn1, conv2, bn2, conv3, bn3,
                dconv=None, dbn=None):
    w1 = conv1[:cin, :cmid]
    w2 = conv2[:9 * cmid, :cmid].reshape(3, 3, cmid, cmid)
    w3 = conv3[:cmid, :cout]
    a1 = bn1[:, :cmid]
    a2 = bn2[:, :cmid]
    a3 = bn3[:, :cout]
    wd = ad = None
    if dconv is not None:
        wd = dconv[:cin, :cout]
        ad = dbn[:, :cout]
    return w1, a1, w2, a2, w3, a3, wd, ad


def kernel(*args):
    (conv1_w, bn1), rest = args[:2], list(args[2:])
    fc_w, fc_b, x = rest[-3], rest[-2], rest[-1]
    rest = rest[:-3]

    blocks = []
    it = iter(rest)
    for (nblk, cmid, cout, _s) in _STAGES:
        stage = []
        for b in range(nblk):
            p = [next(it) for _ in range(8 if b == 0 else 6)]
            stage.append(p)
        blocks.append(stage)

    xh = jnp.transpose(x, (0, 2, 3, 1)).astype(_BF16)
    xh = _stem(xh, conv1_w, bn1)
    xh = _maxpool(xh)

    cin = 64
    for sidx, (nblk, cmid, cout, stride) in enumerate(_STAGES):
        nb = _SUBBATCH[sidx]
        for b in range(nblk):
            p = blocks[sidx][b]
            if b == 0:
                c1, b1, c2, b2, c3, b3, dc, db = p
                prep = _prep_block(cin, cmid, cout, c1, b1, c2, b2, c3, b3,
                                   dc, db)
                s = stride
            else:
                c1, b1, c2, b2, c3, b3 = p
                prep = _prep_block(cout, cmid, cout, c1, b1, c2, b2, c3, b3)
                s = 1
            w1, a1, w2, a2, w3, a3, wd, ad = prep
            xh = _bottleneck(xh, w1, a1, w2, a2, w3, a3, wd, ad, s, nb)
        cin = cout

    return _head(xh, fc_w, fc_b)
